# baseline (speedup 1.0000x reference)
import functools
import math

import jax
import jax.numpy as jnp
from jax import lax
from jax.experimental import pallas as pl
from jax.experimental.pallas import tpu as pltpu
from jax.experimental.pallas import tpu_sc as plsc

D_MODEL = 64
SCALE = math.sqrt(D_MODEL)

_info = plsc.get_sparse_core_info()
NC, NS, L = _info.num_cores, _info.num_subcores, _info.num_lanes
NW = NC * NS

CHUNK = 128
NBUF = 4
LEAD = 2
SBUF = 2


def _make_kernel(B, D):
    assert B % (NW * CHUNK) == 0
    b_per_w = B // NW
    n_chunks = b_per_w // CHUNK
    assert n_chunks % NBUF == 0 and LEAD < NBUF
    mesh = plsc.VectorSubcoreMesh(core_axis_name="c", subcore_axis_name="s")

    @functools.partial(
        pl.kernel,
        out_type=jax.ShapeDtypeStruct((B, D), jnp.float32),
        mesh=mesh,
        scratch_types=[
            pltpu.VMEM((b_per_w,), jnp.int32),
            [pltpu.VMEM((CHUNK, 2 * D), jnp.float32) for _ in range(NBUF)],
            [pltpu.VMEM((CHUNK, D), jnp.float32) for _ in range(SBUF)],
            [pltpu.VMEM((CHUNK,), jnp.int32) for _ in range(NBUF)],
            [pltpu.VMEM((CHUNK + L,), jnp.int32) for _ in range(NBUF)],
            [pltpu.SemaphoreType.DMA for _ in range(NBUF)],
            [pltpu.SemaphoreType.DMA for _ in range(SBUF)],
        ],
        compiler_params=pltpu.CompilerParams(
            use_tc_tiling_on_sc=True, needs_layout_passes=False
        ),
    )
    def k(lut_hbm, idx_hbm, out_hbm, idx_v, gbuf, obuf, ibuf, pbuf, gsem, ssem):
        wid = lax.axis_index("s") * NC + lax.axis_index("c")
        base = wid * b_per_w
        pltpu.sync_copy(idx_hbm.at[pl.ds(base, b_per_w)], idx_v)

        def prep_idx(c, slot):
            for t0 in range(CHUNK // L):
                sl = pl.ds(t0 * L, L)
                raw = idx_v[pl.ds(c * CHUNK + t0 * L, L)]
                ibuf[slot][sl] = lax.shift_right_logical(raw, 1)
                pbuf[slot][sl] = (raw & 1) * D

        def gather_start(slot):
            pltpu.async_copy(lut_hbm.at[ibuf[slot]], gbuf[slot], gsem[slot])

        def gather_wait(slot):
            pltpu.make_async_copy(
                lut_hbm.at[ibuf[slot]], gbuf[slot], gsem[slot]
            ).wait()

        def scatter_start(c, slot):
            pltpu.async_copy(
                obuf[slot],
                out_hbm.at[pl.ds(base + c * CHUNK, CHUNK)],
                ssem[slot],
            )

        def scatter_wait(c, slot):
            pltpu.make_async_copy(
                obuf[slot],
                out_hbm.at[pl.ds(base + c * CHUNK, CHUNK)],
                ssem[slot],
            ).wait()

        def select_scale(gslot, pslot, oslot):
            def t_body(t, carry):
                pv = pbuf[pslot][pl.ds(t, L)]
                par = pv[0]
                for k0 in range(0, D, L):
                    vals = gbuf[gslot][t, pl.ds(par + k0, L)]
                    obuf[oslot][t, pl.ds(k0, L)] = vals * SCALE
                return carry

            lax.fori_loop(0, CHUNK, t_body, 0, unroll=2)

        for b in range(LEAD):
            prep_idx(b, b)
            gather_start(b)

        def group_body(grp, carry):
            for b in range(NBUF):
                c = grp * NBUF + b
                gather_wait(b)
                ts = b % SBUF

                @pl.when(c >= SBUF)
                def _():
                    scatter_wait(c - SBUF, ts)

                select_scale(b, b, ts)
                scatter_start(c, ts)

                h = c + LEAD
                sb = (b + LEAD) % NBUF

                @pl.when(h < n_chunks)
                def _():
                    prep_idx(h, sb)
                    gather_start(sb)

            return carry

        lax.fori_loop(0, n_chunks // NBUF, group_body, 0)

        for c in range(n_chunks - SBUF, n_chunks):
            scatter_wait(c, c % SBUF)

    return k


def kernel(x, lut):
    B = x.shape[0] * x.shape[1]
    lut128 = lut.reshape(lut.shape[0] // 2, 2 * D_MODEL)
    xflat = x.reshape(B).astype(jnp.int32)
    out = _make_kernel(B, D_MODEL)(lut128, xflat)
    return out.reshape(x.shape[0], x.shape[1], D_MODEL)
